# Initial kernel scaffold; baseline (speedup 1.0000x reference)
import functools

import jax
import jax.numpy as jnp
from jax import lax
from jax.experimental import pallas as pl
from jax.experimental.pallas import tpu as pltpu
from jax.experimental.pallas import tpu_sc as plsc

N_NODES = 10000
N_EDGES = 320000
D = 128
C = 10

NC = 2
NS = 16
NW = NC * NS
EPT = N_EDGES // NW
B = 100
NB = EPT // B
RPT = N_NODES // NS
ZCH = 125
NZ = RPT // ZCH


def _sc_aggregate(x, src3, dst3):
  mesh = plsc.VectorSubcoreMesh(core_axis_name="c", subcore_axis_name="s")

  @functools.partial(
      pl.kernel,
      out_type=jax.ShapeDtypeStruct((NC, N_NODES, D), jnp.float32),
      mesh=mesh,
      scratch_types=[
          pltpu.VMEM((NB, B), jnp.int32),
          pltpu.VMEM((NB, B), jnp.int32),
          pltpu.VMEM((B, D), jnp.float32),
          pltpu.VMEM((ZCH, D), jnp.float32),
          pltpu.VMEM_SHARED((N_NODES, D), jnp.float32),
          pltpu.SemaphoreType.DMA,
      ],
  )
  def agg_kernel(x_hbm, src_hbm, dst_hbm, out_hbm,
                 src_v, dst_v, buf, zbuf, acc_sh, sem):
    c = lax.axis_index("c")
    s = lax.axis_index("s")
    wid = c * NS + s

    zero = jnp.zeros((16,), jnp.float32)

    def zrow(i, carry):
      for k in range(D // 16):
        zbuf[i, pl.ds(k * 16, 16)] = zero
      return carry

    lax.fori_loop(0, ZCH, zrow, 0)
    for z in range(NZ):
      pltpu.sync_copy(zbuf, acc_sh.at[pl.ds(s * RPT + z * ZCH, ZCH)])
    plsc.subcore_barrier()

    pltpu.sync_copy(src_hbm.at[wid], src_v)
    pltpu.sync_copy(dst_hbm.at[wid], dst_v)

    def body(j, carry):
      pltpu.async_copy(x_hbm.at[src_v.at[j]], buf, sem).wait()
      pltpu.sync_copy(buf, acc_sh.at[dst_v.at[j]], add=True)
      return carry

    lax.fori_loop(0, NB, body, 0)

    plsc.subcore_barrier()
    pltpu.sync_copy(acc_sh.at[pl.ds(s * RPT, RPT)],
                    out_hbm.at[c, pl.ds(s * RPT, RPT)])

  return agg_kernel(x, src3, dst3)


_BLK = 1000


def _mlp_layer(x, p0, p1, Wa, ba, Wb, bb):
  def body(x_ref, p0_ref, p1_ref, wa, ba_, wb, bb_, y_ref):
    h = x_ref[...] + p0_ref[...] + p1_ref[...]
    a = jnp.maximum(
        jnp.dot(h, wa[...], preferred_element_type=jnp.float32) + ba_[...], 0.0)
    y = jnp.dot(a, wb[...], preferred_element_type=jnp.float32) + bb_[...]
    y_ref[...] = jnp.maximum(y, 0.0)

  row = lambda i: (i, 0)
  full = lambda i: (0, 0)
  return pl.pallas_call(
      body,
      grid=(N_NODES // _BLK,),
      in_specs=[
          pl.BlockSpec((_BLK, D), row),
          pl.BlockSpec((_BLK, D), row),
          pl.BlockSpec((_BLK, D), row),
          pl.BlockSpec((D, D), full),
          pl.BlockSpec((1, D), full),
          pl.BlockSpec((D, D), full),
          pl.BlockSpec((1, D), full),
      ],
      out_specs=pl.BlockSpec((_BLK, D), row),
      out_shape=jax.ShapeDtypeStruct((N_NODES, D), jnp.float32),
  )(x, p0, p1, Wa, ba.reshape(1, D), Wb, bb.reshape(1, D))


def _mlp_layer3_head(x, p0, p1, Wa, ba, Wb, bb, Wc, bc, Wfp, bfp):
  grid_n = N_NODES // _BLK

  def body(x_ref, p0_ref, p1_ref, wa, ba_, wb, bb_, wc, bc_, wf, bf_,
           o_ref, acc):
    i = pl.program_id(0)
    h = x_ref[...] + p0_ref[...] + p1_ref[...]
    a = jnp.maximum(
        jnp.dot(h, wa[...], preferred_element_type=jnp.float32) + ba_[...], 0.0)
    y = jnp.maximum(
        jnp.dot(a, wb[...], preferred_element_type=jnp.float32) + bb_[...], 0.0)
    colsum = jnp.sum(y, axis=0, keepdims=True)

    @pl.when(i == 0)
    def _():
      acc[...] = colsum

    @pl.when(i > 0)
    def _():
      acc[...] = acc[...] + colsum

    @pl.when(i == grid_n - 1)
    def _():
      pooled = acc[...] * (1.0 / N_NODES)
      r = jnp.dot(pooled, wc[...], preferred_element_type=jnp.float32) + bc_[...]
      o_ref[...] = jnp.dot(r, wf[...], preferred_element_type=jnp.float32) + bf_[...]

  row = lambda i: (i, 0)
  full = lambda i: (0, 0)
  return pl.pallas_call(
      body,
      grid=(grid_n,),
      in_specs=[
          pl.BlockSpec((_BLK, D), row),
          pl.BlockSpec((_BLK, D), row),
          pl.BlockSpec((_BLK, D), row),
          pl.BlockSpec((D, D), full),
          pl.BlockSpec((1, D), full),
          pl.BlockSpec((D, D), full),
          pl.BlockSpec((1, D), full),
          pl.BlockSpec((D, D), full),
          pl.BlockSpec((1, D), full),
          pl.BlockSpec((D, D), full),
          pl.BlockSpec((1, D), full),
      ],
      out_specs=pl.BlockSpec((1, D), full),
      out_shape=jax.ShapeDtypeStruct((1, D), jnp.float32),
      scratch_shapes=[pltpu.VMEM((1, D), jnp.float32)],
  )(x, p0, p1, Wa, ba.reshape(1, D), Wb, bb.reshape(1, D),
    Wc, bc.reshape(1, D), Wfp, bfp.reshape(1, D))


def kernel(x, edge_index, W1a, b1a, W1b, b1b, W2a, b2a, W2b, b2b,
           W3a, b3a, W3b, b3b, Wc, bc, Wf, bf):
  ei = edge_index.astype(jnp.int32)
  src3 = ei[0].reshape(NW, NB, B)
  dst3 = ei[1].reshape(NW, NB, B)

  Wfp = jnp.zeros((D, D), jnp.float32).at[:, :C].set(Wf)
  bfp = jnp.zeros((D,), jnp.float32).at[:C].set(bf)

  h = x
  p = _sc_aggregate(h, src3, dst3)
  h = _mlp_layer(h, p[0], p[1], W1a, b1a, W1b, b1b)
  p = _sc_aggregate(h, src3, dst3)
  h = _mlp_layer(h, p[0], p[1], W2a, b2a, W2b, b2b)
  p = _sc_aggregate(h, src3, dst3)
  out = _mlp_layer3_head(h, p[0], p[1], W3a, b3a, W3b, b3b, Wc, bc, Wfp, bfp)

  return (out[:, :C], edge_index)


# speedup vs baseline: 7.8818x; 7.8818x over previous
import functools

import jax
import jax.numpy as jnp
from jax import lax
from jax.experimental import pallas as pl
from jax.experimental.pallas import tpu as pltpu
from jax.experimental.pallas import tpu_sc as plsc

N_NODES = 10000
N_EDGES = 320000
D = 128
C = 10

NC = 2
NS = 16
NW = NC * NS
EPT = N_EDGES // NW
B = 100
NB = EPT // B
N_PAD = 10240
RPT = N_PAD // NS
ZCH = 64
NZ = RPT // ZCH


def _sc_aggregate(x, src3, dst3):
  mesh = plsc.VectorSubcoreMesh(core_axis_name="c", subcore_axis_name="s")

  @functools.partial(
      pl.kernel,
      out_type=jax.ShapeDtypeStruct((NC, N_PAD, D), jnp.float32),
      mesh=mesh,
      scratch_types=[
          pltpu.VMEM((NB, B), jnp.int32),
          pltpu.VMEM((NB, B), jnp.int32),
          pltpu.VMEM((B, D), jnp.float32),
          pltpu.VMEM((ZCH, D), jnp.float32),
          pltpu.VMEM_SHARED((N_PAD, D), jnp.float32),
          pltpu.SemaphoreType.DMA,
      ],
  )
  def agg_kernel(x_hbm, src_hbm, dst_hbm, out_hbm,
                 src_v, dst_v, buf, zbuf, acc_sh, sem):
    c = lax.axis_index("c")
    s = lax.axis_index("s")
    wid = c * NS + s

    zero = jnp.zeros((16,), jnp.float32)

    def zrow(i, carry):
      for k in range(D // 16):
        zbuf[i, pl.ds(k * 16, 16)] = zero
      return carry

    lax.fori_loop(0, ZCH, zrow, 0)
    for z in range(NZ):
      pltpu.sync_copy(zbuf, acc_sh.at[pl.ds(s * RPT + z * ZCH, ZCH)])
    plsc.subcore_barrier()

    pltpu.sync_copy(src_hbm.at[wid], src_v)
    pltpu.sync_copy(dst_hbm.at[wid], dst_v)

    def body(j, carry):
      pltpu.async_copy(x_hbm.at[src_v.at[j]], buf, sem).wait()
      pltpu.sync_copy(buf, acc_sh.at[dst_v.at[j]], add=True)
      return carry

    lax.fori_loop(0, NB, body, 0)

    plsc.subcore_barrier()
    pltpu.sync_copy(acc_sh.at[pl.ds(s * RPT, RPT)],
                    out_hbm.at[c, pl.ds(s * RPT, RPT)])

  return agg_kernel(x, src3, dst3)


_BLK = 1000


def _mlp_layer(x, p0, p1, Wa, ba, Wb, bb):
  def body(x_ref, p0_ref, p1_ref, wa, ba_, wb, bb_, y_ref):
    h = x_ref[...] + p0_ref[0] + p1_ref[0]
    a = jnp.maximum(
        jnp.dot(h, wa[...], preferred_element_type=jnp.float32) + ba_[...], 0.0)
    y = jnp.dot(a, wb[...], preferred_element_type=jnp.float32) + bb_[...]
    y_ref[...] = jnp.maximum(y, 0.0)

  row = lambda i: (i, 0)
  full = lambda i: (0, 0)
  return pl.pallas_call(
      body,
      grid=(N_NODES // _BLK,),
      in_specs=[
          pl.BlockSpec((_BLK, D), row),
          pl.BlockSpec((1, _BLK, D), lambda i: (0, i, 0)),
          pl.BlockSpec((1, _BLK, D), lambda i: (1, i, 0)),
          pl.BlockSpec((D, D), full),
          pl.BlockSpec((1, D), full),
          pl.BlockSpec((D, D), full),
          pl.BlockSpec((1, D), full),
      ],
      out_specs=pl.BlockSpec((_BLK, D), row),
      out_shape=jax.ShapeDtypeStruct((N_NODES, D), jnp.float32),
  )(x, p0, p1, Wa, ba.reshape(1, D), Wb, bb.reshape(1, D))


def _mlp_layer3_head(x, p0, p1, Wa, ba, Wb, bb, Wc, bc, Wfp, bfp):
  grid_n = N_NODES // _BLK

  def body(x_ref, p0_ref, p1_ref, wa, ba_, wb, bb_, wc, bc_, wf, bf_,
           o_ref, acc):
    i = pl.program_id(0)
    h = x_ref[...] + p0_ref[0] + p1_ref[0]
    a = jnp.maximum(
        jnp.dot(h, wa[...], preferred_element_type=jnp.float32) + ba_[...], 0.0)
    y = jnp.maximum(
        jnp.dot(a, wb[...], preferred_element_type=jnp.float32) + bb_[...], 0.0)
    colsum = jnp.sum(y, axis=0, keepdims=True)

    @pl.when(i == 0)
    def _():
      acc[...] = colsum

    @pl.when(i > 0)
    def _():
      acc[...] = acc[...] + colsum

    @pl.when(i == grid_n - 1)
    def _():
      pooled = acc[...] * (1.0 / N_NODES)
      r = jnp.dot(pooled, wc[...], preferred_element_type=jnp.float32) + bc_[...]
      o_ref[...] = jnp.dot(r, wf[...], preferred_element_type=jnp.float32) + bf_[...]

  row = lambda i: (i, 0)
  full = lambda i: (0, 0)
  return pl.pallas_call(
      body,
      grid=(grid_n,),
      in_specs=[
          pl.BlockSpec((_BLK, D), row),
          pl.BlockSpec((1, _BLK, D), lambda i: (0, i, 0)),
          pl.BlockSpec((1, _BLK, D), lambda i: (1, i, 0)),
          pl.BlockSpec((D, D), full),
          pl.BlockSpec((1, D), full),
          pl.BlockSpec((D, D), full),
          pl.BlockSpec((1, D), full),
          pl.BlockSpec((D, D), full),
          pl.BlockSpec((1, D), full),
          pl.BlockSpec((D, D), full),
          pl.BlockSpec((1, D), full),
      ],
      out_specs=pl.BlockSpec((1, D), full),
      out_shape=jax.ShapeDtypeStruct((1, D), jnp.float32),
      scratch_shapes=[pltpu.VMEM((1, D), jnp.float32)],
  )(x, p0, p1, Wa, ba.reshape(1, D), Wb, bb.reshape(1, D),
    Wc, bc.reshape(1, D), Wfp, bfp.reshape(1, D))


def kernel(x, edge_index, W1a, b1a, W1b, b1b, W2a, b2a, W2b, b2b,
           W3a, b3a, W3b, b3b, Wc, bc, Wf, bf):
  ei = edge_index.astype(jnp.int32)
  src3 = ei[0].reshape(NW, NB, B)
  dst3 = ei[1].reshape(NW, NB, B)

  Wfp = jnp.zeros((D, D), jnp.float32).at[:, :C].set(Wf)
  bfp = jnp.zeros((D,), jnp.float32).at[:C].set(bf)

  h = x
  p = _sc_aggregate(h, src3, dst3)
  h = _mlp_layer(h, p, p, W1a, b1a, W1b, b1b)
  p = _sc_aggregate(h, src3, dst3)
  h = _mlp_layer(h, p, p, W2a, b2a, W2b, b2b)
  p = _sc_aggregate(h, src3, dst3)
  out = _mlp_layer3_head(h, p, p, W3a, b3a, W3b, b3b, Wc, bc, Wfp, bfp)

  return (out[:, :C], edge_index)
